# Initial kernel scaffold; baseline (speedup 1.0000x reference)
import jax
import jax.numpy as jnp
from jax.experimental import pallas as pl


def kernel(x, edge_index, W1_l, b1_l, W1_r, W2_l, b2_l, W2_r):
    raise NotImplementedError("write your pallas kernel here")


# speedup vs baseline: 3.8218x; 3.8218x over previous
import functools

import jax
import jax.numpy as jnp
from jax import lax
from jax.experimental import pallas as pl
from jax.experimental.pallas import tpu as pltpu
from jax.experimental.pallas import tpu_sc as plsc

N = 10000
D = 128

NC = 2
NS = 16
NW = NC * NS

C = 128
ROWS_PER_SUB = 632
N_PAD = NS * ROWS_PER_SUB


def _agg_body(iters, feat, srcr, dstr, zrow,
              agg_out, srcv, dstv, msgv, aggs, sem):
    c = lax.axis_index("c")
    s = lax.axis_index("s")
    wid = c * NS + s
    pltpu.sync_copy(zrow, aggs.at[pl.ds(s * ROWS_PER_SUB, ROWS_PER_SUB)])
    plsc.subcore_barrier()

    epw = iters * C
    base = wid * epw

    @pl.loop(0, iters)
    def _(i):
        off = base + i * C
        pltpu.sync_copy(srcr.at[pl.ds(off, C)], srcv)
        pltpu.sync_copy(dstr.at[pl.ds(off, C)], dstv)
        pltpu.async_copy(feat.at[srcv], msgv, sem).wait()
        pltpu.sync_copy(msgv, aggs.at[dstv], add=True)

    plsc.subcore_barrier()
    r0 = s * ROWS_PER_SUB
    pltpu.sync_copy(aggs.at[pl.ds(r0, ROWS_PER_SUB)],
                    agg_out.at[c, pl.ds(r0, ROWS_PER_SUB)])


def _make_agg(iters):
    mesh = plsc.VectorSubcoreMesh(core_axis_name="c", subcore_axis_name="s")
    return pl.kernel(
        functools.partial(_agg_body, iters),
        out_type=jax.ShapeDtypeStruct((NC, N_PAD, D), jnp.float32),
        mesh=mesh,
        scratch_types=[
            pltpu.VMEM((C,), jnp.int32),
            pltpu.VMEM((C,), jnp.int32),
            pltpu.VMEM((C, D), jnp.float32),
            pltpu.VMEM_SHARED((N_PAD, D), jnp.float32),
            pltpu.SemaphoreType.DMA,
        ],
    )


def _cnt_body(iters, dstr, zrow, ones_h,
              cnt_out, dstv, onesv, cnts):
    c = lax.axis_index("c")
    s = lax.axis_index("s")
    wid = c * NS + s
    pltpu.sync_copy(zrow, cnts.at[pl.ds(s * ROWS_PER_SUB, ROWS_PER_SUB)])
    pltpu.sync_copy(ones_h, onesv)
    plsc.subcore_barrier()

    epw = iters * C
    base = wid * epw

    @pl.loop(0, iters)
    def _(i):
        pltpu.sync_copy(dstr.at[pl.ds(base + i * C, C)], dstv)
        pltpu.sync_copy(onesv, cnts.at[dstv], add=True)

    plsc.subcore_barrier()
    r0 = s * ROWS_PER_SUB
    pltpu.sync_copy(cnts.at[pl.ds(r0, ROWS_PER_SUB)],
                    cnt_out.at[c, pl.ds(r0, ROWS_PER_SUB)])


def _make_cnt(iters):
    mesh = plsc.VectorSubcoreMesh(core_axis_name="c", subcore_axis_name="s")
    return pl.kernel(
        functools.partial(_cnt_body, iters),
        out_type=jax.ShapeDtypeStruct((NC, N_PAD, D), jnp.float32),
        mesh=mesh,
        scratch_types=[
            pltpu.VMEM((C,), jnp.int32),
            pltpu.VMEM((C, D), jnp.float32),
            pltpu.VMEM_SHARED((N_PAD, D), jnp.float32),
        ],
    )


def _dense_body(relu, p0, p1, c0, c1, xr, wl, wr, b, out):
    cnt = jnp.clip(c0[...] + c1[...], 1.0, None)
    mean = (p0[...] + p1[...]) / cnt
    acc = jnp.dot(mean, wl[...], preferred_element_type=jnp.float32)
    acc = acc + jnp.dot(xr[...], wr[...], preferred_element_type=jnp.float32)
    acc = acc + b[...]
    if relu:
        acc = jnp.maximum(acc, 0.0)
    out[...] = acc


def _dense(p0, p1, c0, c1, x, wl, wr, b, relu):
    R = 1000
    grid = (N // R,)
    row_spec = pl.BlockSpec((R, D), lambda i: (i, 0))
    cnt_spec = pl.BlockSpec((R, 1), lambda i: (i, 0))
    w_spec = pl.BlockSpec((D, D), lambda i: (0, 0))
    b_spec = pl.BlockSpec((1, D), lambda i: (0, 0))
    return pl.pallas_call(
        functools.partial(_dense_body, relu),
        grid=grid,
        in_specs=[row_spec, row_spec, cnt_spec, cnt_spec, row_spec,
                  w_spec, w_spec, b_spec],
        out_specs=row_spec,
        out_shape=jax.ShapeDtypeStruct((N, D), jnp.float32),
    )(p0, p1, c0, c1, x, wl, wr, b)


def kernel(x, edge_index, W1_l, b1_l, W1_r, W2_l, b2_l, W2_r):
    E = edge_index.shape[1]
    chunk = NW * C
    iters = -(-E // chunk)
    e_pad = iters * chunk
    src = edge_index[0].astype(jnp.int32)
    dst = edge_index[1].astype(jnp.int32)
    pad = e_pad - E
    if pad:
        src = jnp.concatenate([src, jnp.zeros((pad,), jnp.int32)])
        dst = jnp.concatenate([dst, jnp.full((pad,), N, jnp.int32)])

    zrow = jnp.zeros((ROWS_PER_SUB, D), jnp.float32)
    ones_h = jnp.ones((C, D), jnp.float32)

    agg = _make_agg(iters)
    cntk = _make_cnt(iters)
    b1 = b1_l.reshape(1, D)
    b2 = b2_l.reshape(1, D)

    cnt = cntk(dst, zrow, ones_h)
    c0 = cnt[0, :N, 0:1]
    c1 = cnt[1, :N, 0:1]
    a1 = agg(x, src, dst, zrow)
    h = _dense(a1[0, :N], a1[1, :N], c0, c1, x, W1_l, W1_r, b1, relu=True)
    a2 = agg(h, src, dst, zrow)
    out = _dense(a2[0, :N], a2[1, :N], c0, c1, h, W2_l, W2_r, b2, relu=False)
    return out
